# baseline (speedup 1.0000x reference)
import jax
import jax.numpy as jnp
import numpy as np
from jax.experimental import pallas as pl

EMB = 64
HEADS = 2
NG = 256


def _pe(x, d):
    xf = x.astype(jnp.float32)
    freq = 1.0 / (10000.0 ** (jnp.arange(0, d, 2, dtype=jnp.float32) / d))
    ang = xf[:, None] * freq[None, :]
    pe = jnp.concatenate([jnp.sin(ang), jnp.cos(ang)], axis=-1)
    return jnp.where((x == -1)[:, None], 0.0, pe)


def _gln(x, batch, g):
    cnt = jax.ops.segment_sum(jnp.ones(x.shape[0], jnp.float32), batch, num_segments=g)
    cnt = jnp.maximum(cnt, 1.0)
    d = x.shape[1]
    mean = jax.ops.segment_sum(x.sum(axis=1), batch, num_segments=g) / (cnt * d)
    xc = x - mean[batch][:, None]
    var = jax.ops.segment_sum((xc * xc).sum(axis=1), batch, num_segments=g) / (cnt * d)
    return xc / jnp.sqrt(var[batch][:, None] + 1e-5)


def _segsoft(logits, seg, num_segments):
    m = jax.ops.segment_max(logits, seg, num_segments=num_segments)
    m = jnp.where(jnp.isfinite(m), m, 0.0)
    ex = jnp.exp(logits - m[seg])
    s = jax.ops.segment_sum(ex, seg, num_segments=num_segments)
    return ex / (s[seg] + 1e-16)


def _mlp2(x, W1, b1, W2, b2):
    return jax.nn.leaky_relu(x @ W1 + b1, 0.01) @ W2 + b2


def _layer(x, src, dst, e, batch_aug, g, p):
    n = x.shape[0]
    x_norm = _gln(x, batch_aug, g)
    msg = jax.nn.relu(x_norm[src] + e) + 1e-7
    agg = jax.ops.segment_sum(msg, dst, num_segments=n)
    gen_out = (x_norm + agg) @ p['W_gen'] + p['b_gen']
    x2 = jnp.concatenate([x_norm, gen_out], axis=1)
    dh = EMB
    q = (x2 @ p['Wq'] + p['bq']).reshape(n, HEADS, dh)
    ke = (e @ p['We'] + p['be']).reshape(-1, HEADS, dh)
    k = (x2 @ p['Wk'] + p['bk']).reshape(n, HEADS, dh)[src] + ke
    v = (x2 @ p['Wv'] + p['bv']).reshape(n, HEADS, dh)[src] + ke
    alpha = (q[dst] * k).sum(-1) / jnp.sqrt(float(dh))
    alpha = _segsoft(alpha, dst, n)
    out = jax.ops.segment_sum(alpha[:, :, None] * v, dst, num_segments=n).reshape(n, HEADS * dh)
    out = out + x2 @ p['Wskip'] + p['bskip']
    l_h = out @ p['Wl'] + p['bl']
    h = _gln(l_h, batch_aug, g)
    ff = _mlp2(h, p['W1'], p['b1'], p['W2'], p['b2'])
    return x + ff


def _heads_kernel(glob_ref, ne_ref,
                  iw1, ib1, iw2, ib2,
                  nw1, nb1, nw2, nb2,
                  ew1, eb1, ew2, eb2,
                  tw1, tb1, tw2, tb2,
                  o_init, o_nodelv, o_stop, o_tgt):
    def m2(x, W1, b1, W2, b2):
        h = jnp.dot(x, W1[...], preferred_element_type=jnp.float32) + b1[...]
        h = jnp.where(h >= 0, h, 0.01 * h)
        return jnp.dot(h, W2[...], preferred_element_type=jnp.float32) + b2[...]
    g = glob_ref[...]
    o_init[...] = m2(g, iw1, ib1, iw2, ib2)
    o_nodelv[...] = m2(g, nw1, nb1, nw2, nb2)
    o_stop[...] = m2(g, ew1, eb1, ew2, eb2)
    o_tgt[...] = m2(ne_ref[...], tw1, tb1, tw2, tb2)


def _heads(glob, ne_emb, params):
    b2 = lambda b: b.reshape(1, -1)
    outs = pl.pallas_call(
        _heads_kernel,
        out_shape=[
            jax.ShapeDtypeStruct((NG, 16), jnp.float32),
            jax.ShapeDtypeStruct((NG, 65), jnp.float32),
            jax.ShapeDtypeStruct((NG, 1), jnp.float32),
            jax.ShapeDtypeStruct((ne_emb.shape[0], 4), jnp.float32),
        ],
    )(glob, ne_emb,
      params['init_W1'], b2(params['init_b1']), params['init_W2'], b2(params['init_b2']),
      params['nodelv_W1'], b2(params['nodelv_b1']), params['nodelv_W2'], b2(params['nodelv_b2']),
      params['e1_W1'], b2(params['e1_b1']), params['e1_W2'], b2(params['e1_b2']),
      params['e2_W1'], b2(params['e2_b1']), params['e2_W2'], b2(params['e2_b2']))
    return outs


def kernel(params, node_type, node_state_type, frontier_order, edge_type, edge_index, batch, non_edge_index):
    n = node_type.shape[0]
    g = NG
    x = params['node_type_emb'][node_type] + params['node_state_emb'][node_state_type] \
        + _pe(frontier_order, EMB)
    e = params['edge_type_emb'][edge_type]
    cond = jnp.broadcast_to(params['virtual_emb'][0], (g, EMB))
    x_aug = jnp.concatenate([x, cond], axis=0)
    u = jnp.arange(n)
    v = batch + n
    aug_ei = jnp.concatenate([edge_index, jnp.stack([u, v]), jnp.stack([v, u])], axis=1)
    e_p = jnp.zeros((2 * n, EMB), jnp.float32).at[:, 0].set(1.0)
    aug_e = jnp.concatenate([e, e_p], axis=0)
    n_aug = n + g
    loop = jnp.arange(n_aug)
    aug_ei = jnp.concatenate([aug_ei, jnp.stack([loop, loop])], axis=1)
    loop_attr = jnp.broadcast_to(aug_e.mean(axis=0), (n_aug, EMB))
    aug_e = jnp.concatenate([aug_e, loop_attr], axis=0)
    aug_batch = jnp.concatenate([batch, jnp.arange(g)], axis=0)
    src, dst = aug_ei[0], aug_ei[1]
    h = x_aug
    for p in params['layers']:
        h = _layer(h, src, dst, aug_e, aug_batch, g, p)
    n_emb = h[:n]
    v_emb = h[n:]
    cnt = jnp.maximum(jax.ops.segment_sum(jnp.ones(n, jnp.float32), batch, num_segments=g), 1.0)
    glob = jax.ops.segment_sum(n_emb, batch, num_segments=g) / cnt[:, None] + v_emb
    ne_emb = n_emb[non_edge_index[0]] + n_emb[non_edge_index[1]]
    i = g // 3
    j = 2 * g // 3
    init_f, nodelv_f, stop_f, tgt_f = _heads(glob, ne_emb, params)
    init_logits = init_f[:i]
    nodelv_logits = nodelv_f[i:j]
    stop_logits = stop_f[j:].reshape(-1)
    tgt_logits = tgt_f.reshape(-1)
    edgelv_logits = jnp.concatenate([stop_logits, tgt_logits], axis=0)
    return jnp.concatenate([init_logits.reshape(-1), nodelv_logits.reshape(-1), edgelv_logits], axis=0)


# speedup vs baseline: 2.6863x; 2.6863x over previous
import jax
import jax.numpy as jnp
import numpy as np
from jax.experimental import pallas as pl

EMB = 64
HEADS = 2
NG = 256
N = 10000
NAUG = N + NG
NP = 10368
CH = 256
CH2 = 512
F32 = jnp.float32


def _f32(x):
    return x.astype(F32)


def _iota(shape, dim):
    return jax.lax.broadcasted_iota(jnp.int32, shape, dim)


def _onehot_cols(idx_col, width):
    return (idx_col == _iota((idx_col.shape[0], width), 1)).astype(F32)


def _seg_ln(x, batch_col):
    S = _onehot_cols(batch_col, NG)
    ones = jnp.ones((x.shape[0], 1), F32)
    cd = (((0,), (0,)), ((), ()))
    cnt = jax.lax.dot_general(S, ones, cd, preferred_element_type=F32)
    cnt = jnp.maximum(cnt, 1.0)
    sums = jax.lax.dot_general(S, x, cd, preferred_element_type=F32)
    mean_g = jnp.sum(sums, axis=1, keepdims=True) / (cnt * x.shape[1])
    mean_n = jnp.dot(S, mean_g, preferred_element_type=F32)
    xc = x - mean_n
    rs = jnp.sum(xc * xc, axis=1, keepdims=True)
    var_g = jax.lax.dot_general(S, rs, cd, preferred_element_type=F32) / (cnt * x.shape[1])
    var_n = jnp.dot(S, var_g, preferred_element_type=F32)
    return xc / jnp.sqrt(var_n + 1e-5)


def _setup_kernel(nt_ref, ns_ref, fo_ref, et_ref, ntab_ref, stab_ref, etab_ref,
                  x0_ref, tab6_ref):
    nt = nt_ref[...]
    ns = ns_ref[...]
    x0 = jnp.dot(_onehot_cols(nt, 24), ntab_ref[...], preferred_element_type=F32)
    x0 = x0 + jnp.dot(_onehot_cols(ns, 8), stab_ref[...], preferred_element_type=F32)
    fo = fo_ref[...]
    fof = _f32(fo)
    ii = _f32(_iota((1, EMB // 2), 1))
    freq = jnp.exp(ii * (-np.log(10000.0) * 2.0 / EMB))
    ang = fof * freq
    pe = jnp.concatenate([jnp.sin(ang), jnp.cos(ang)], axis=1)
    x0_ref[...] = x0 + jnp.where(fo == -1, 0.0, pe)
    etab = etab_ref[...]
    et = et_ref[...]
    ep_row = (_iota((1, EMB), 1) == 0).astype(F32)
    rid = _iota((8, 1), 0)
    loop_row = 20000.0 * ep_row
    for t in range(4):
        cnt_t = jnp.sum((et == t).astype(F32))
        row_t = jnp.sum(etab * (rid == t).astype(F32), axis=0, keepdims=True)
        loop_row = loop_row + cnt_t * row_t
    loop_row = loop_row / 180000.0
    tab6 = etab * (rid < 4).astype(F32)
    tab6 = tab6 + (rid == 4).astype(F32) * ep_row
    tab6 = tab6 + (rid == 5).astype(F32) * loop_row
    tab6_ref[...] = tab6


def _ln_kernel(x_ref, b_ref, o_ref):
    o_ref[...] = _seg_ln(x_ref[...], b_ref[...])


def _gen_kernel(xn_ref, tab6_ref, src_ref, dst_ref, code_ref, agg_ref):
    @pl.when(pl.program_id(0) == 0)
    def _():
        agg_ref[...] = jnp.zeros_like(agg_ref)
    ms = _onehot_cols(src_ref[0], NP)
    md = _onehot_cols(dst_ref[0], NP)
    xg = jnp.dot(ms, xn_ref[...], preferred_element_type=F32)
    xg = xg + jnp.dot(_onehot_cols(code_ref[0], 8), tab6_ref[...],
                      preferred_element_type=F32)
    msg = jnp.maximum(xg, 0.0) + 1e-7
    cd = (((0,), (0,)), ((), ()))
    agg_ref[...] += jax.lax.dot_general(md, msg, cd, preferred_element_type=F32)


def _proj_kernel(xn_ref, agg_ref, tab6_ref,
                 wg_ref, bg_ref, wq_ref, bq_ref, wk_ref, bk_ref,
                 wv_ref, bv_ref, ws_ref, bs_ref, we_ref, be_ref,
                 qn_ref, kn_ref, vn_ref, skip_ref, ketab_ref):
    xn = xn_ref[...]
    gen = jnp.dot(xn + agg_ref[...], wg_ref[...], preferred_element_type=F32) + bg_ref[...]
    x2 = jnp.concatenate([xn, gen], axis=1)
    qn_ref[...] = jnp.dot(x2, wq_ref[...], preferred_element_type=F32) + bq_ref[...]
    kn_ref[...] = jnp.dot(x2, wk_ref[...], preferred_element_type=F32) + bk_ref[...]
    vn_ref[...] = jnp.dot(x2, wv_ref[...], preferred_element_type=F32) + bv_ref[...]
    skip_ref[...] = jnp.dot(x2, ws_ref[...], preferred_element_type=F32) + bs_ref[...]
    ketab_ref[...] = jnp.dot(tab6_ref[...], we_ref[...], preferred_element_type=F32) + be_ref[...]


def _att_kernel(qn_ref, kn_ref, vn_ref, ketab_ref, src_ref, dst_ref, code_ref, att_ref):
    @pl.when(pl.program_id(0) == 0)
    def _():
        att_ref[...] = jnp.zeros_like(att_ref)
    ms = _onehot_cols(src_ref[0], NP)
    md = _onehot_cols(dst_ref[0], NP)
    ke = jnp.dot(_onehot_cols(code_ref[0], 8), ketab_ref[...],
                 preferred_element_type=F32)
    k_e = jnp.dot(ms, kn_ref[...], preferred_element_type=F32) + ke
    v_e = jnp.dot(ms, vn_ref[...], preferred_element_type=F32) + ke
    q_e = jnp.dot(md, qn_ref[...], preferred_element_type=F32)
    l1 = jnp.sum(q_e[:, :EMB] * k_e[:, :EMB], axis=1, keepdims=True) * 0.125
    l2 = jnp.sum(q_e[:, EMB:] * k_e[:, EMB:], axis=1, keepdims=True) * 0.125
    ex1 = jnp.exp(l1)
    ex2 = jnp.exp(l2)
    scat = jnp.concatenate([ex1 * v_e[:, :EMB], ex2 * v_e[:, EMB:], ex1, ex2], axis=1)
    cd = (((0,), (0,)), ((), ()))
    att_ref[...] += jax.lax.dot_general(md, scat, cd, preferred_element_type=F32)


def _post_kernel(att_ref, skip_ref, xres_ref, b_ref,
                 wl_ref, bl_ref, w1_ref, b1_ref, w2_ref, b2_ref, o_ref):
    att = att_ref[...]
    o1 = att[:, 0:EMB] / (att[:, 2 * EMB:2 * EMB + 1] + 1e-16)
    o2 = att[:, EMB:2 * EMB] / (att[:, 2 * EMB + 1:2 * EMB + 2] + 1e-16)
    out = jnp.concatenate([o1, o2], axis=1) + skip_ref[...]
    l_h = jnp.dot(out, wl_ref[...], preferred_element_type=F32) + bl_ref[...]
    h = _seg_ln(l_h, b_ref[...])
    hh = jnp.dot(h, w1_ref[...], preferred_element_type=F32) + b1_ref[...]
    hh = jnp.where(hh >= 0, hh, 0.01 * hh)
    ff = jnp.dot(hh, w2_ref[...], preferred_element_type=F32) + b2_ref[...]
    o_ref[...] = xres_ref[...] + ff


def _glob_heads_kernel(h_ref, breal_ref,
                       iw1, ib1, iw2, ib2, nw1, nb1, nw2, nb2, ew1, eb1, ew2, eb2,
                       o_init, o_nodelv, o_stop):
    S = _onehot_cols(breal_ref[...], NG)
    h = h_ref[...]
    ones = jnp.ones((NP, 1), F32)
    cd = (((0,), (0,)), ((), ()))
    cnt = jnp.maximum(jax.lax.dot_general(S, ones, cd, preferred_element_type=F32), 1.0)
    sums = jax.lax.dot_general(S, h, cd, preferred_element_type=F32)
    glob = sums / cnt + h_ref[N:NAUG, :]

    def m2(x, W1, b1, W2, b2):
        t = jnp.dot(x, W1[...], preferred_element_type=F32) + b1[...]
        t = jnp.where(t >= 0, t, 0.01 * t)
        return jnp.dot(t, W2[...], preferred_element_type=F32) + b2[...]
    o_init[...] = m2(glob, iw1, ib1, iw2, ib2)
    o_nodelv[...] = m2(glob, nw1, nb1, nw2, nb2)
    o_stop[...] = m2(glob, ew1, eb1, ew2, eb2)


def _ne_kernel(h_ref, a_ref, b_ref, tw1, tb1, tw2, tb2, o_ref):
    m = _onehot_cols(a_ref[0], NP) + _onehot_cols(b_ref[0], NP)
    ne = jnp.dot(m, h_ref[...], preferred_element_type=F32)
    t = jnp.dot(ne, tw1[...], preferred_element_type=F32) + tb1[...]
    t = jnp.where(t >= 0, t, 0.01 * t)
    o_ref[0] = jnp.dot(t, tw2[...], preferred_element_type=F32) + tb2[...]


def _pad_rows(x, rows):
    return jnp.pad(x, ((0, rows - x.shape[0]), (0, 0)))


def _col3(idx, chunk, pad_val):
    e = idx.shape[0]
    nch = -(-e // chunk)
    p = jnp.full((nch * chunk,), pad_val, jnp.int32).at[:e].set(idx.astype(jnp.int32))
    return p.reshape(nch, chunk, 1), nch


def _edge_call(kfn, n_out_lanes, nch, full_ins, idx_ins):
    grid = (nch,)
    in_specs = ([pl.BlockSpec(a.shape, lambda i: (0,) * a.ndim) for a in full_ins]
                + [pl.BlockSpec((1, a.shape[1], 1), lambda i: (i, 0, 0)) for a in idx_ins])
    return pl.pallas_call(
        kfn,
        grid=grid,
        in_specs=in_specs,
        out_specs=pl.BlockSpec((NP, n_out_lanes), lambda i: (0, 0)),
        out_shape=jax.ShapeDtypeStruct((NP, n_out_lanes), F32),
    )(*full_ins, *idx_ins)


def kernel(params, node_type, node_state_type, frontier_order, edge_type, edge_index, batch, non_edge_index):
    b2 = lambda b: b.reshape(1, -1)
    i32 = lambda a: a.astype(jnp.int32)

    ntab = _pad_rows(params['node_type_emb'], 24)
    stab = _pad_rows(params['node_state_emb'], 8)
    etab = _pad_rows(params['edge_type_emb'], 8)
    x0, tab6 = pl.pallas_call(
        _setup_kernel,
        out_shape=[jax.ShapeDtypeStruct((N, EMB), F32),
                   jax.ShapeDtypeStruct((8, EMB), F32)],
    )(i32(node_type).reshape(N, 1), i32(node_state_type).reshape(N, 1),
      i32(frontier_order).reshape(N, 1),
      jnp.pad(i32(edge_type), (0, 160768 - edge_type.shape[0]),
              constant_values=100).reshape(1256, 128),
      ntab, stab, etab)

    cond = jnp.broadcast_to(params['virtual_emb'][0], (NG, EMB))
    x_aug = _pad_rows(jnp.concatenate([x0, cond], axis=0), NP)
    u = jnp.arange(N, dtype=jnp.int32)
    v = i32(batch) + N
    loop = jnp.arange(NAUG, dtype=jnp.int32)
    src = jnp.concatenate([i32(edge_index[0]), u, v, loop])
    dst = jnp.concatenate([i32(edge_index[1]), v, u, loop])
    codes = jnp.concatenate([i32(edge_type), jnp.full((2 * N,), 4, jnp.int32),
                             jnp.full((NAUG,), 5, jnp.int32)])
    src3, nch = _col3(src, CH, NP)
    dst3, _ = _col3(dst, CH, NP)
    cod3, _ = _col3(codes, CH, 0)
    batch_col = jnp.concatenate([i32(batch), jnp.arange(NG, dtype=jnp.int32),
                                 jnp.full((NP - NAUG,), NG, jnp.int32)]).reshape(NP, 1)

    ln_call = pl.pallas_call(
        _ln_kernel, out_shape=jax.ShapeDtypeStruct((NP, EMB), F32))
    proj_call = pl.pallas_call(
        _proj_kernel,
        out_shape=[jax.ShapeDtypeStruct((NP, 2 * EMB), F32)] * 4
        + [jax.ShapeDtypeStruct((8, 2 * EMB), F32)])
    post_call = pl.pallas_call(
        _post_kernel, out_shape=jax.ShapeDtypeStruct((NP, EMB), F32))

    h = x_aug
    for p in params['layers']:
        xn = ln_call(h, batch_col)
        agg = _edge_call(_gen_kernel, EMB, nch, [xn, tab6], [src3, dst3, cod3])
        qn, kn, vn, skip, ketab = proj_call(
            xn, agg, tab6,
            p['W_gen'], b2(p['b_gen']), p['Wq'], b2(p['bq']), p['Wk'], b2(p['bk']),
            p['Wv'], b2(p['bv']), p['Wskip'], b2(p['bskip']), p['We'], b2(p['be']))
        att = _edge_call(_att_kernel, 2 * EMB + 2, nch,
                         [qn, kn, vn, ketab], [src3, dst3, cod3])
        h = post_call(att, skip, h, batch_col,
                      p['Wl'], b2(p['bl']), p['W1'], b2(p['b1']), p['W2'], b2(p['b2']))

    breal_col = jnp.concatenate([i32(batch), jnp.full((NP - N,), NG, jnp.int32)]).reshape(NP, 1)
    init_f, nodelv_f, stop_f = pl.pallas_call(
        _glob_heads_kernel,
        out_shape=[jax.ShapeDtypeStruct((NG, 16), F32),
                   jax.ShapeDtypeStruct((NG, 65), F32),
                   jax.ShapeDtypeStruct((NG, 1), F32)],
    )(h, breal_col,
      params['init_W1'], b2(params['init_b1']), params['init_W2'], b2(params['init_b2']),
      params['nodelv_W1'], b2(params['nodelv_b1']), params['nodelv_W2'], b2(params['nodelv_b2']),
      params['e1_W1'], b2(params['e1_b1']), params['e1_W2'], b2(params['e1_b2']))

    a3, nne = _col3(i32(non_edge_index[0]), CH2, NP)
    bb3, _ = _col3(i32(non_edge_index[1]), CH2, NP)
    tgt_f = pl.pallas_call(
        _ne_kernel,
        grid=(nne,),
        in_specs=[pl.BlockSpec((NP, EMB), lambda i: (0, 0)),
                  pl.BlockSpec((1, CH2, 1), lambda i: (i, 0, 0)),
                  pl.BlockSpec((1, CH2, 1), lambda i: (i, 0, 0)),
                  pl.BlockSpec((EMB, EMB), lambda i: (0, 0)),
                  pl.BlockSpec((1, EMB), lambda i: (0, 0)),
                  pl.BlockSpec((EMB, 4), lambda i: (0, 0)),
                  pl.BlockSpec((1, 4), lambda i: (0, 0))],
        out_specs=pl.BlockSpec((1, CH2, 4), lambda i: (i, 0, 0)),
        out_shape=jax.ShapeDtypeStruct((nne, CH2, 4), F32),
    )(h, a3, bb3, params['e2_W1'], b2(params['e2_b1']), params['e2_W2'], b2(params['e2_b2']))

    i = NG // 3
    j = 2 * NG // 3
    nne_total = non_edge_index.shape[1]
    return jnp.concatenate([
        init_f[:i].reshape(-1), nodelv_f[i:j].reshape(-1), stop_f[j:].reshape(-1),
        tgt_f.reshape(-1, 4)[:nne_total].reshape(-1)], axis=0)


# speedup vs baseline: 2.8043x; 1.0439x over previous
import jax
import jax.numpy as jnp
import numpy as np
from jax.experimental import pallas as pl
from jax.experimental.pallas import tpu as pltpu

EMB = 64
HEADS = 2
NG = 256
N = 10000
NAUG = N + NG
NP = 10368
CH = 512
CH2 = 512
F32 = jnp.float32


def _f32(x):
    return x.astype(F32)


def _iota(shape, dim):
    return jax.lax.broadcasted_iota(jnp.int32, shape, dim)


def _onehot_cols(idx_col, width):
    return (idx_col == _iota((idx_col.shape[0], width), 1)).astype(F32)


def _seg_ln(x, batch_col):
    S = _onehot_cols(batch_col, NG)
    ones = jnp.ones((x.shape[0], 1), F32)
    cd = (((0,), (0,)), ((), ()))
    cnt = jax.lax.dot_general(S, ones, cd, preferred_element_type=F32)
    cnt = jnp.maximum(cnt, 1.0)
    sums = jax.lax.dot_general(S, x, cd, preferred_element_type=F32)
    mean_g = jnp.sum(sums, axis=1, keepdims=True) / (cnt * x.shape[1])
    mean_n = jnp.dot(S, mean_g, preferred_element_type=F32)
    xc = x - mean_n
    rs = jnp.sum(xc * xc, axis=1, keepdims=True)
    var_g = jax.lax.dot_general(S, rs, cd, preferred_element_type=F32) / (cnt * x.shape[1])
    var_n = jnp.dot(S, var_g, preferred_element_type=F32)
    return xc / jnp.sqrt(var_n + 1e-5)


def _setup_kernel(nt_ref, ns_ref, fo_ref, et_ref, ntab_ref, stab_ref, etab_ref,
                  x0_ref, tab6_ref):
    nt = nt_ref[...]
    ns = ns_ref[...]
    x0 = jnp.dot(_onehot_cols(nt, 24), ntab_ref[...], preferred_element_type=F32)
    x0 = x0 + jnp.dot(_onehot_cols(ns, 8), stab_ref[...], preferred_element_type=F32)
    fo = fo_ref[...]
    fof = _f32(fo)
    ii = _f32(_iota((1, EMB // 2), 1))
    freq = jnp.exp(ii * (-np.log(10000.0) * 2.0 / EMB))
    ang = fof * freq
    pe = jnp.concatenate([jnp.sin(ang), jnp.cos(ang)], axis=1)
    x0_ref[...] = x0 + jnp.where(fo == -1, 0.0, pe)
    etab = etab_ref[...]
    et = et_ref[...]
    ep_row = (_iota((1, EMB), 1) == 0).astype(F32)
    rid = _iota((8, 1), 0)
    loop_row = 20000.0 * ep_row
    for t in range(4):
        cnt_t = jnp.sum((et == t).astype(F32))
        row_t = jnp.sum(etab * (rid == t).astype(F32), axis=0, keepdims=True)
        loop_row = loop_row + cnt_t * row_t
    loop_row = loop_row / 180000.0
    tab6 = etab * (rid < 4).astype(F32)
    tab6 = tab6 + (rid == 4).astype(F32) * ep_row
    tab6 = tab6 + (rid == 5).astype(F32) * loop_row
    tab6_ref[...] = tab6


def _ln_kernel(x_ref, b_ref, o_ref):
    o_ref[...] = _seg_ln(x_ref[...], b_ref[...])


def _gen_kernel(xn_ref, tab6_ref, src_ref, dst_ref, code_ref, agg_ref):
    @pl.when(pl.program_id(0) == 0)
    def _():
        agg_ref[...] = jnp.zeros_like(agg_ref)
    ms = _onehot_cols(src_ref[0], NP)
    md = _onehot_cols(dst_ref[0], NP)
    xg = jnp.dot(ms, xn_ref[...], preferred_element_type=F32)
    xg = xg + jnp.dot(_onehot_cols(code_ref[0], 8), tab6_ref[...],
                      preferred_element_type=F32)
    msg = jnp.maximum(xg, 0.0) + 1e-7
    cd = (((0,), (0,)), ((), ()))
    agg_ref[...] += jax.lax.dot_general(md, msg, cd, preferred_element_type=F32)


def _proj_kernel(xn_ref, agg_ref, tab6_ref,
                 wg_ref, bg_ref, wq_ref, bq_ref, wk_ref, bk_ref,
                 wv_ref, bv_ref, ws_ref, bs_ref, we_ref, be_ref,
                 qn_ref, kn_ref, vn_ref, skip_ref, ketab_ref):
    xn = xn_ref[...]
    gen = jnp.dot(xn + agg_ref[...], wg_ref[...], preferred_element_type=F32) + bg_ref[...]
    x2 = jnp.concatenate([xn, gen], axis=1)
    qn_ref[...] = jnp.dot(x2, wq_ref[...], preferred_element_type=F32) + bq_ref[...]
    kn_ref[...] = jnp.dot(x2, wk_ref[...], preferred_element_type=F32) + bk_ref[...]
    vn_ref[...] = jnp.dot(x2, wv_ref[...], preferred_element_type=F32) + bv_ref[...]
    skip_ref[...] = jnp.dot(x2, ws_ref[...], preferred_element_type=F32) + bs_ref[...]
    ketab_ref[...] = jnp.dot(tab6_ref[...], we_ref[...], preferred_element_type=F32) + be_ref[...]


def _att_kernel(qn_ref, kn_ref, vn_ref, ketab_ref, src_ref, dst_ref, code_ref, att_ref):
    @pl.when(pl.program_id(0) == 0)
    def _():
        att_ref[...] = jnp.zeros_like(att_ref)
    ms = _onehot_cols(src_ref[0], NP)
    md = _onehot_cols(dst_ref[0], NP)
    ke = jnp.dot(_onehot_cols(code_ref[0], 8), ketab_ref[...],
                 preferred_element_type=F32)
    k_e = jnp.dot(ms, kn_ref[...], preferred_element_type=F32) + ke
    v_e = jnp.dot(ms, vn_ref[...], preferred_element_type=F32) + ke
    q_e = jnp.dot(md, qn_ref[...], preferred_element_type=F32)
    l1 = jnp.sum(q_e[:, :EMB] * k_e[:, :EMB], axis=1, keepdims=True) * 0.125
    l2 = jnp.sum(q_e[:, EMB:] * k_e[:, EMB:], axis=1, keepdims=True) * 0.125
    ex1 = jnp.exp(l1)
    ex2 = jnp.exp(l2)
    scat = jnp.concatenate([ex1 * v_e[:, :EMB], ex2 * v_e[:, EMB:], ex1, ex2], axis=1)
    cd = (((0,), (0,)), ((), ()))
    att_ref[...] += jax.lax.dot_general(md, scat, cd, preferred_element_type=F32)


def _post_kernel(att_ref, skip_ref, xres_ref, b_ref,
                 wl_ref, bl_ref, w1_ref, b1_ref, w2_ref, b2_ref, o_ref):
    att = att_ref[...]
    o1 = att[:, 0:EMB] / (att[:, 2 * EMB:2 * EMB + 1] + 1e-16)
    o2 = att[:, EMB:2 * EMB] / (att[:, 2 * EMB + 1:2 * EMB + 2] + 1e-16)
    out = jnp.concatenate([o1, o2], axis=1) + skip_ref[...]
    l_h = jnp.dot(out, wl_ref[...], preferred_element_type=F32) + bl_ref[...]
    h = _seg_ln(l_h, b_ref[...])
    hh = jnp.dot(h, w1_ref[...], preferred_element_type=F32) + b1_ref[...]
    hh = jnp.where(hh >= 0, hh, 0.01 * hh)
    ff = jnp.dot(hh, w2_ref[...], preferred_element_type=F32) + b2_ref[...]
    o_ref[...] = xres_ref[...] + ff


def _glob_heads_kernel(h_ref, breal_ref,
                       iw1, ib1, iw2, ib2, nw1, nb1, nw2, nb2, ew1, eb1, ew2, eb2,
                       o_init, o_nodelv, o_stop):
    S = _onehot_cols(breal_ref[...], NG)
    h = h_ref[...]
    ones = jnp.ones((NP, 1), F32)
    cd = (((0,), (0,)), ((), ()))
    cnt = jnp.maximum(jax.lax.dot_general(S, ones, cd, preferred_element_type=F32), 1.0)
    sums = jax.lax.dot_general(S, h, cd, preferred_element_type=F32)
    glob = sums / cnt + h_ref[N:NAUG, :]

    def m2(x, W1, b1, W2, b2):
        t = jnp.dot(x, W1[...], preferred_element_type=F32) + b1[...]
        t = jnp.where(t >= 0, t, 0.01 * t)
        return jnp.dot(t, W2[...], preferred_element_type=F32) + b2[...]
    o_init[...] = m2(glob, iw1, ib1, iw2, ib2)
    o_nodelv[...] = m2(glob, nw1, nb1, nw2, nb2)
    o_stop[...] = m2(glob, ew1, eb1, ew2, eb2)


def _ne_kernel(h_ref, a_ref, b_ref, tw1, tb1, tw2, tb2, o_ref):
    m = _onehot_cols(a_ref[0], NP) + _onehot_cols(b_ref[0], NP)
    ne = jnp.dot(m, h_ref[...], preferred_element_type=F32)
    t = jnp.dot(ne, tw1[...], preferred_element_type=F32) + tb1[...]
    t = jnp.where(t >= 0, t, 0.01 * t)
    o_ref[0] = jnp.dot(t, tw2[...], preferred_element_type=F32) + tb2[...]


def _pad_rows(x, rows):
    return jnp.pad(x, ((0, rows - x.shape[0]), (0, 0)))


def _col3(idx, chunk, pad_val):
    e = idx.shape[0]
    nch = -(-e // chunk)
    p = jnp.full((nch * chunk,), pad_val, jnp.int32).at[:e].set(idx.astype(jnp.int32))
    return p.reshape(nch, chunk, 1), nch


def _edge_call(kfn, n_out_lanes, nch, full_ins, idx_ins):
    grid = (nch,)
    in_specs = ([pl.BlockSpec(a.shape, lambda i: (0,) * a.ndim) for a in full_ins]
                + [pl.BlockSpec((1, a.shape[1], 1), lambda i: (i, 0, 0)) for a in idx_ins])
    return pl.pallas_call(
        kfn,
        grid=grid,
        in_specs=in_specs,
        out_specs=pl.BlockSpec((NP, n_out_lanes), lambda i: (0, 0)),
        out_shape=jax.ShapeDtypeStruct((NP, n_out_lanes), F32),
        compiler_params=pltpu.CompilerParams(vmem_limit_bytes=120 * 1024 * 1024),
    )(*full_ins, *idx_ins)


def kernel(params, node_type, node_state_type, frontier_order, edge_type, edge_index, batch, non_edge_index):
    b2 = lambda b: b.reshape(1, -1)
    i32 = lambda a: a.astype(jnp.int32)

    ntab = _pad_rows(params['node_type_emb'], 24)
    stab = _pad_rows(params['node_state_emb'], 8)
    etab = _pad_rows(params['edge_type_emb'], 8)
    x0, tab6 = pl.pallas_call(
        _setup_kernel,
        out_shape=[jax.ShapeDtypeStruct((N, EMB), F32),
                   jax.ShapeDtypeStruct((8, EMB), F32)],
    )(i32(node_type).reshape(N, 1), i32(node_state_type).reshape(N, 1),
      i32(frontier_order).reshape(N, 1),
      jnp.pad(i32(edge_type), (0, 160768 - edge_type.shape[0]),
              constant_values=100).reshape(1256, 128),
      ntab, stab, etab)

    cond = jnp.broadcast_to(params['virtual_emb'][0], (NG, EMB))
    x_aug = _pad_rows(jnp.concatenate([x0, cond], axis=0), NP)
    u = jnp.arange(N, dtype=jnp.int32)
    v = i32(batch) + N
    loop = jnp.arange(NAUG, dtype=jnp.int32)
    src = jnp.concatenate([i32(edge_index[0]), u, v, loop])
    dst = jnp.concatenate([i32(edge_index[1]), v, u, loop])
    codes = jnp.concatenate([i32(edge_type), jnp.full((2 * N,), 4, jnp.int32),
                             jnp.full((NAUG,), 5, jnp.int32)])
    src3, nch = _col3(src, CH, NP)
    dst3, _ = _col3(dst, CH, NP)
    cod3, _ = _col3(codes, CH, 0)
    batch_col = jnp.concatenate([i32(batch), jnp.arange(NG, dtype=jnp.int32),
                                 jnp.full((NP - NAUG,), NG, jnp.int32)]).reshape(NP, 1)

    ln_call = pl.pallas_call(
        _ln_kernel, out_shape=jax.ShapeDtypeStruct((NP, EMB), F32))
    proj_call = pl.pallas_call(
        _proj_kernel,
        out_shape=[jax.ShapeDtypeStruct((NP, 2 * EMB), F32)] * 4
        + [jax.ShapeDtypeStruct((8, 2 * EMB), F32)])
    post_call = pl.pallas_call(
        _post_kernel, out_shape=jax.ShapeDtypeStruct((NP, EMB), F32))

    h = x_aug
    for p in params['layers']:
        xn = ln_call(h, batch_col)
        agg = _edge_call(_gen_kernel, EMB, nch, [xn, tab6], [src3, dst3, cod3])
        qn, kn, vn, skip, ketab = proj_call(
            xn, agg, tab6,
            p['W_gen'], b2(p['b_gen']), p['Wq'], b2(p['bq']), p['Wk'], b2(p['bk']),
            p['Wv'], b2(p['bv']), p['Wskip'], b2(p['bskip']), p['We'], b2(p['be']))
        att = _edge_call(_att_kernel, 2 * EMB + 2, nch,
                         [qn, kn, vn, ketab], [src3, dst3, cod3])
        h = post_call(att, skip, h, batch_col,
                      p['Wl'], b2(p['bl']), p['W1'], b2(p['b1']), p['W2'], b2(p['b2']))

    breal_col = jnp.concatenate([i32(batch), jnp.full((NP - N,), NG, jnp.int32)]).reshape(NP, 1)
    init_f, nodelv_f, stop_f = pl.pallas_call(
        _glob_heads_kernel,
        out_shape=[jax.ShapeDtypeStruct((NG, 16), F32),
                   jax.ShapeDtypeStruct((NG, 65), F32),
                   jax.ShapeDtypeStruct((NG, 1), F32)],
    )(h, breal_col,
      params['init_W1'], b2(params['init_b1']), params['init_W2'], b2(params['init_b2']),
      params['nodelv_W1'], b2(params['nodelv_b1']), params['nodelv_W2'], b2(params['nodelv_b2']),
      params['e1_W1'], b2(params['e1_b1']), params['e1_W2'], b2(params['e1_b2']))

    a3, nne = _col3(i32(non_edge_index[0]), CH2, NP)
    bb3, _ = _col3(i32(non_edge_index[1]), CH2, NP)
    tgt_f = pl.pallas_call(
        _ne_kernel,
        grid=(nne,),
        in_specs=[pl.BlockSpec((NP, EMB), lambda i: (0, 0)),
                  pl.BlockSpec((1, CH2, 1), lambda i: (i, 0, 0)),
                  pl.BlockSpec((1, CH2, 1), lambda i: (i, 0, 0)),
                  pl.BlockSpec((EMB, EMB), lambda i: (0, 0)),
                  pl.BlockSpec((1, EMB), lambda i: (0, 0)),
                  pl.BlockSpec((EMB, 4), lambda i: (0, 0)),
                  pl.BlockSpec((1, 4), lambda i: (0, 0))],
        out_specs=pl.BlockSpec((1, CH2, 4), lambda i: (i, 0, 0)),
        out_shape=jax.ShapeDtypeStruct((nne, CH2, 4), F32),
    )(h, a3, bb3, params['e2_W1'], b2(params['e2_b1']), params['e2_W2'], b2(params['e2_b2']))

    i = NG // 3
    j = 2 * NG // 3
    nne_total = non_edge_index.shape[1]
    return jnp.concatenate([
        init_f[:i].reshape(-1), nodelv_f[i:j].reshape(-1), stop_f[j:].reshape(-1),
        tgt_f.reshape(-1, 4)[:nne_total].reshape(-1)], axis=0)


# speedup vs baseline: 2.8321x; 1.0099x over previous
import jax
import jax.numpy as jnp
import numpy as np
from jax.experimental import pallas as pl
from jax.experimental.pallas import tpu as pltpu

EMB = 64
HEADS = 2
NG = 256
N = 10000
NAUG = N + NG
NP = 10368
CH = 512
CH2 = 512
F32 = jnp.float32


def _f32(x):
    return x.astype(F32)


def _iota(shape, dim):
    return jax.lax.broadcasted_iota(jnp.int32, shape, dim)


def _onehot_cols(idx_col, width, dtype=F32):
    return (idx_col == _iota((idx_col.shape[0], width), 1)).astype(dtype)


def _seg_ln(x, batch_col):
    S = _onehot_cols(batch_col, NG)
    ones = jnp.ones((x.shape[0], 1), F32)
    cd = (((0,), (0,)), ((), ()))
    cnt = jax.lax.dot_general(S, ones, cd, preferred_element_type=F32)
    cnt = jnp.maximum(cnt, 1.0)
    sums = jax.lax.dot_general(S, x, cd, preferred_element_type=F32)
    mean_g = jnp.sum(sums, axis=1, keepdims=True) / (cnt * x.shape[1])
    mean_n = jnp.dot(S, mean_g, preferred_element_type=F32)
    xc = x - mean_n
    rs = jnp.sum(xc * xc, axis=1, keepdims=True)
    var_g = jax.lax.dot_general(S, rs, cd, preferred_element_type=F32) / (cnt * x.shape[1])
    var_n = jnp.dot(S, var_g, preferred_element_type=F32)
    return xc / jnp.sqrt(var_n + 1e-5)


def _setup_kernel(nt_ref, ns_ref, fo_ref, et_ref, ntab_ref, stab_ref, etab_ref,
                  x0_ref, tab6_ref):
    nt = nt_ref[...]
    ns = ns_ref[...]
    x0 = jnp.dot(_onehot_cols(nt, 24), ntab_ref[...], preferred_element_type=F32)
    x0 = x0 + jnp.dot(_onehot_cols(ns, 8), stab_ref[...], preferred_element_type=F32)
    fo = fo_ref[...]
    fof = _f32(fo)
    ii = _f32(_iota((1, EMB // 2), 1))
    freq = jnp.exp(ii * (-np.log(10000.0) * 2.0 / EMB))
    ang = fof * freq
    pe = jnp.concatenate([jnp.sin(ang), jnp.cos(ang)], axis=1)
    x0_ref[...] = x0 + jnp.where(fo == -1, 0.0, pe)
    etab = etab_ref[...]
    et = et_ref[...]
    ep_row = (_iota((1, EMB), 1) == 0).astype(F32)
    rid = _iota((8, 1), 0)
    loop_row = 20000.0 * ep_row
    for t in range(4):
        cnt_t = jnp.sum((et == t).astype(F32))
        row_t = jnp.sum(etab * (rid == t).astype(F32), axis=0, keepdims=True)
        loop_row = loop_row + cnt_t * row_t
    loop_row = loop_row / 180000.0
    tab6 = etab * (rid < 4).astype(F32)
    tab6 = tab6 + (rid == 4).astype(F32) * ep_row
    tab6 = tab6 + (rid == 5).astype(F32) * loop_row
    tab6_ref[...] = tab6


BF16 = jnp.bfloat16


def _ln_kernel(x_ref, b_ref, o_ref, obf_ref):
    xn = _seg_ln(x_ref[...], b_ref[...])
    o_ref[...] = xn
    obf_ref[...] = xn.astype(BF16)


def _gen_kernel(xn_ref, tab6_ref, src_ref, dst_ref, code_ref, agg_ref):
    @pl.when(pl.program_id(0) == 0)
    def _():
        agg_ref[...] = jnp.zeros_like(agg_ref)
    ms = _onehot_cols(src_ref[0], NP, BF16)
    md = _onehot_cols(dst_ref[0], NP, BF16)
    xg = jnp.dot(ms, xn_ref[...], preferred_element_type=F32)
    xg = xg + jnp.dot(_onehot_cols(code_ref[0], 8), tab6_ref[...],
                      preferred_element_type=F32)
    msg = (jnp.maximum(xg, 0.0) + 1e-7).astype(BF16)
    cd = (((0,), (0,)), ((), ()))
    agg_ref[...] += jax.lax.dot_general(md, msg, cd, preferred_element_type=F32)


def _proj_kernel(xn_ref, agg_ref, tab6_ref,
                 wg_ref, bg_ref, wq_ref, bq_ref, wk_ref, bk_ref,
                 wv_ref, bv_ref, ws_ref, bs_ref, we_ref, be_ref,
                 qn_ref, kn_ref, vn_ref, skip_ref, ketab_ref):
    xn = xn_ref[...]
    gen = jnp.dot(xn + agg_ref[...], wg_ref[...], preferred_element_type=F32) + bg_ref[...]
    x2 = jnp.concatenate([xn, gen], axis=1)
    qn_ref[...] = (jnp.dot(x2, wq_ref[...], preferred_element_type=F32) + bq_ref[...]).astype(BF16)
    kn_ref[...] = (jnp.dot(x2, wk_ref[...], preferred_element_type=F32) + bk_ref[...]).astype(BF16)
    vn_ref[...] = (jnp.dot(x2, wv_ref[...], preferred_element_type=F32) + bv_ref[...]).astype(BF16)
    skip_ref[...] = jnp.dot(x2, ws_ref[...], preferred_element_type=F32) + bs_ref[...]
    ketab_ref[...] = jnp.dot(tab6_ref[...], we_ref[...], preferred_element_type=F32) + be_ref[...]


def _att_kernel(qn_ref, kn_ref, vn_ref, ketab_ref, src_ref, dst_ref, code_ref, att_ref):
    @pl.when(pl.program_id(0) == 0)
    def _():
        att_ref[...] = jnp.zeros_like(att_ref)
    ms = _onehot_cols(src_ref[0], NP, BF16)
    md = _onehot_cols(dst_ref[0], NP, BF16)
    ke = jnp.dot(_onehot_cols(code_ref[0], 8), ketab_ref[...],
                 preferred_element_type=F32)
    k_e = jnp.dot(ms, kn_ref[...], preferred_element_type=F32) + ke
    v_e = jnp.dot(ms, vn_ref[...], preferred_element_type=F32) + ke
    q_e = jnp.dot(md, qn_ref[...], preferred_element_type=F32)
    l1 = jnp.sum(q_e[:, :EMB] * k_e[:, :EMB], axis=1, keepdims=True) * 0.125
    l2 = jnp.sum(q_e[:, EMB:] * k_e[:, EMB:], axis=1, keepdims=True) * 0.125
    ex1 = jnp.exp(l1)
    ex2 = jnp.exp(l2)
    scat = jnp.concatenate([ex1 * v_e[:, :EMB], ex2 * v_e[:, EMB:], ex1, ex2],
                           axis=1).astype(BF16)
    cd = (((0,), (0,)), ((), ()))
    att_ref[...] += jax.lax.dot_general(md, scat, cd, preferred_element_type=F32)


def _post_kernel(att_ref, skip_ref, xres_ref, b_ref,
                 wl_ref, bl_ref, w1_ref, b1_ref, w2_ref, b2_ref, o_ref):
    att = att_ref[...]
    o1 = att[:, 0:EMB] / (att[:, 2 * EMB:2 * EMB + 1] + 1e-16)
    o2 = att[:, EMB:2 * EMB] / (att[:, 2 * EMB + 1:2 * EMB + 2] + 1e-16)
    out = jnp.concatenate([o1, o2], axis=1) + skip_ref[...]
    l_h = jnp.dot(out, wl_ref[...], preferred_element_type=F32) + bl_ref[...]
    h = _seg_ln(l_h, b_ref[...])
    hh = jnp.dot(h, w1_ref[...], preferred_element_type=F32) + b1_ref[...]
    hh = jnp.where(hh >= 0, hh, 0.01 * hh)
    ff = jnp.dot(hh, w2_ref[...], preferred_element_type=F32) + b2_ref[...]
    o_ref[...] = xres_ref[...] + ff


def _glob_heads_kernel(h_ref, breal_ref,
                       iw1, ib1, iw2, ib2, nw1, nb1, nw2, nb2, ew1, eb1, ew2, eb2,
                       o_init, o_nodelv, o_stop):
    S = _onehot_cols(breal_ref[...], NG)
    h = h_ref[...]
    ones = jnp.ones((NP, 1), F32)
    cd = (((0,), (0,)), ((), ()))
    cnt = jnp.maximum(jax.lax.dot_general(S, ones, cd, preferred_element_type=F32), 1.0)
    sums = jax.lax.dot_general(S, h, cd, preferred_element_type=F32)
    glob = sums / cnt + h_ref[N:NAUG, :]

    def m2(x, W1, b1, W2, b2):
        t = jnp.dot(x, W1[...], preferred_element_type=F32) + b1[...]
        t = jnp.where(t >= 0, t, 0.01 * t)
        return jnp.dot(t, W2[...], preferred_element_type=F32) + b2[...]
    o_init[...] = m2(glob, iw1, ib1, iw2, ib2)
    o_nodelv[...] = m2(glob, nw1, nb1, nw2, nb2)
    o_stop[...] = m2(glob, ew1, eb1, ew2, eb2)


def _ne_kernel(h_ref, a_ref, b_ref, tw1, tb1, tw2, tb2, o_ref):
    m = _onehot_cols(a_ref[0], NP, BF16) + _onehot_cols(b_ref[0], NP, BF16)
    ne = jnp.dot(m, h_ref[...], preferred_element_type=F32)
    t = jnp.dot(ne, tw1[...], preferred_element_type=F32) + tb1[...]
    t = jnp.where(t >= 0, t, 0.01 * t)
    o_ref[0] = jnp.dot(t, tw2[...], preferred_element_type=F32) + tb2[...]


def _pad_rows(x, rows):
    return jnp.pad(x, ((0, rows - x.shape[0]), (0, 0)))


def _col3(idx, chunk, pad_val):
    e = idx.shape[0]
    nch = -(-e // chunk)
    p = jnp.full((nch * chunk,), pad_val, jnp.int32).at[:e].set(idx.astype(jnp.int32))
    return p.reshape(nch, chunk, 1), nch


def _edge_call(kfn, n_out_lanes, nch, full_ins, idx_ins):
    grid = (nch,)
    in_specs = ([pl.BlockSpec(a.shape, lambda i: (0,) * a.ndim) for a in full_ins]
                + [pl.BlockSpec((1, a.shape[1], 1), lambda i: (i, 0, 0)) for a in idx_ins])
    return pl.pallas_call(
        kfn,
        grid=grid,
        in_specs=in_specs,
        out_specs=pl.BlockSpec((NP, n_out_lanes), lambda i: (0, 0)),
        out_shape=jax.ShapeDtypeStruct((NP, n_out_lanes), F32),
        compiler_params=pltpu.CompilerParams(vmem_limit_bytes=120 * 1024 * 1024),
    )(*full_ins, *idx_ins)


def kernel(params, node_type, node_state_type, frontier_order, edge_type, edge_index, batch, non_edge_index):
    b2 = lambda b: b.reshape(1, -1)
    i32 = lambda a: a.astype(jnp.int32)

    ntab = _pad_rows(params['node_type_emb'], 24)
    stab = _pad_rows(params['node_state_emb'], 8)
    etab = _pad_rows(params['edge_type_emb'], 8)
    x0, tab6 = pl.pallas_call(
        _setup_kernel,
        out_shape=[jax.ShapeDtypeStruct((N, EMB), F32),
                   jax.ShapeDtypeStruct((8, EMB), F32)],
    )(i32(node_type).reshape(N, 1), i32(node_state_type).reshape(N, 1),
      i32(frontier_order).reshape(N, 1),
      jnp.pad(i32(edge_type), (0, 160768 - edge_type.shape[0]),
              constant_values=100).reshape(1256, 128),
      ntab, stab, etab)

    cond = jnp.broadcast_to(params['virtual_emb'][0], (NG, EMB))
    x_aug = _pad_rows(jnp.concatenate([x0, cond], axis=0), NP)
    u = jnp.arange(N, dtype=jnp.int32)
    v = i32(batch) + N
    loop = jnp.arange(NAUG, dtype=jnp.int32)
    src = jnp.concatenate([i32(edge_index[0]), u, v, loop])
    dst = jnp.concatenate([i32(edge_index[1]), v, u, loop])
    codes = jnp.concatenate([i32(edge_type), jnp.full((2 * N,), 4, jnp.int32),
                             jnp.full((NAUG,), 5, jnp.int32)])
    src3, nch = _col3(src, CH, NP)
    dst3, _ = _col3(dst, CH, NP)
    cod3, _ = _col3(codes, CH, 0)
    batch_col = jnp.concatenate([i32(batch), jnp.arange(NG, dtype=jnp.int32),
                                 jnp.full((NP - NAUG,), NG, jnp.int32)]).reshape(NP, 1)

    ln_call = pl.pallas_call(
        _ln_kernel, out_shape=[jax.ShapeDtypeStruct((NP, EMB), F32),
                               jax.ShapeDtypeStruct((NP, EMB), BF16)])
    proj_call = pl.pallas_call(
        _proj_kernel,
        out_shape=[jax.ShapeDtypeStruct((NP, 2 * EMB), BF16)] * 3
        + [jax.ShapeDtypeStruct((NP, 2 * EMB), F32),
           jax.ShapeDtypeStruct((8, 2 * EMB), F32)])
    post_call = pl.pallas_call(
        _post_kernel, out_shape=jax.ShapeDtypeStruct((NP, EMB), F32))

    h = x_aug
    for p in params['layers']:
        xn, xn_bf = ln_call(h, batch_col)
        agg = _edge_call(_gen_kernel, EMB, nch, [xn_bf, tab6], [src3, dst3, cod3])
        qn, kn, vn, skip, ketab = proj_call(
            xn, agg, tab6,
            p['W_gen'], b2(p['b_gen']), p['Wq'], b2(p['bq']), p['Wk'], b2(p['bk']),
            p['Wv'], b2(p['bv']), p['Wskip'], b2(p['bskip']), p['We'], b2(p['be']))
        att = _edge_call(_att_kernel, 2 * EMB + 2, nch,
                         [qn, kn, vn, ketab], [src3, dst3, cod3])
        h = post_call(att, skip, h, batch_col,
                      p['Wl'], b2(p['bl']), p['W1'], b2(p['b1']), p['W2'], b2(p['b2']))

    breal_col = jnp.concatenate([i32(batch), jnp.full((NP - N,), NG, jnp.int32)]).reshape(NP, 1)
    init_f, nodelv_f, stop_f = pl.pallas_call(
        _glob_heads_kernel,
        out_shape=[jax.ShapeDtypeStruct((NG, 16), F32),
                   jax.ShapeDtypeStruct((NG, 65), F32),
                   jax.ShapeDtypeStruct((NG, 1), F32)],
    )(h, breal_col,
      params['init_W1'], b2(params['init_b1']), params['init_W2'], b2(params['init_b2']),
      params['nodelv_W1'], b2(params['nodelv_b1']), params['nodelv_W2'], b2(params['nodelv_b2']),
      params['e1_W1'], b2(params['e1_b1']), params['e1_W2'], b2(params['e1_b2']))

    a3, nne = _col3(i32(non_edge_index[0]), CH2, NP)
    bb3, _ = _col3(i32(non_edge_index[1]), CH2, NP)
    tgt_f = pl.pallas_call(
        _ne_kernel,
        grid=(nne,),
        in_specs=[pl.BlockSpec((NP, EMB), lambda i: (0, 0)),
                  pl.BlockSpec((1, CH2, 1), lambda i: (i, 0, 0)),
                  pl.BlockSpec((1, CH2, 1), lambda i: (i, 0, 0)),
                  pl.BlockSpec((EMB, EMB), lambda i: (0, 0)),
                  pl.BlockSpec((1, EMB), lambda i: (0, 0)),
                  pl.BlockSpec((EMB, 4), lambda i: (0, 0)),
                  pl.BlockSpec((1, 4), lambda i: (0, 0))],
        out_specs=pl.BlockSpec((1, CH2, 4), lambda i: (i, 0, 0)),
        out_shape=jax.ShapeDtypeStruct((nne, CH2, 4), F32),
    )(h.astype(BF16), a3, bb3,
      params['e2_W1'], b2(params['e2_b1']), params['e2_W2'], b2(params['e2_b2']))

    i = NG // 3
    j = 2 * NG // 3
    nne_total = non_edge_index.shape[1]
    return jnp.concatenate([
        init_f[:i].reshape(-1), nodelv_f[i:j].reshape(-1), stop_f[j:].reshape(-1),
        tgt_f.reshape(-1, 4)[:nne_total].reshape(-1)], axis=0)


# speedup vs baseline: 2.8921x; 1.0212x over previous
import jax
import jax.numpy as jnp
import numpy as np
from jax.experimental import pallas as pl
from jax.experimental.pallas import tpu as pltpu

EMB = 64
HEADS = 2
NG = 256
N = 10000
NAUG = N + NG
NP = 10368
CH = 1024
CH2 = 512
F32 = jnp.float32


def _f32(x):
    return x.astype(F32)


def _iota(shape, dim):
    return jax.lax.broadcasted_iota(jnp.int32, shape, dim)


def _onehot_cols(idx_col, width, dtype=F32):
    return (idx_col == _iota((idx_col.shape[0], width), 1)).astype(dtype)


def _seg_ln(x, batch_col):
    S = _onehot_cols(batch_col, NG)
    ones = jnp.ones((x.shape[0], 1), F32)
    cd = (((0,), (0,)), ((), ()))
    cnt = jax.lax.dot_general(S, ones, cd, preferred_element_type=F32)
    cnt = jnp.maximum(cnt, 1.0)
    sums = jax.lax.dot_general(S, x, cd, preferred_element_type=F32)
    mean_g = jnp.sum(sums, axis=1, keepdims=True) / (cnt * x.shape[1])
    mean_n = jnp.dot(S, mean_g, preferred_element_type=F32)
    xc = x - mean_n
    rs = jnp.sum(xc * xc, axis=1, keepdims=True)
    var_g = jax.lax.dot_general(S, rs, cd, preferred_element_type=F32) / (cnt * x.shape[1])
    var_n = jnp.dot(S, var_g, preferred_element_type=F32)
    return xc / jnp.sqrt(var_n + 1e-5)


def _setup_kernel(nt_ref, ns_ref, fo_ref, et_ref, ntab_ref, stab_ref, etab_ref,
                  x0_ref, tab6_ref):
    nt = nt_ref[...]
    ns = ns_ref[...]
    x0 = jnp.dot(_onehot_cols(nt, 24), ntab_ref[...], preferred_element_type=F32)
    x0 = x0 + jnp.dot(_onehot_cols(ns, 8), stab_ref[...], preferred_element_type=F32)
    fo = fo_ref[...]
    fof = _f32(fo)
    ii = _f32(_iota((1, EMB // 2), 1))
    freq = jnp.exp(ii * (-np.log(10000.0) * 2.0 / EMB))
    ang = fof * freq
    pe = jnp.concatenate([jnp.sin(ang), jnp.cos(ang)], axis=1)
    x0_ref[...] = x0 + jnp.where(fo == -1, 0.0, pe)
    etab = etab_ref[...]
    et = et_ref[...]
    ep_row = (_iota((1, EMB), 1) == 0).astype(F32)
    rid = _iota((8, 1), 0)
    loop_row = 20000.0 * ep_row
    for t in range(4):
        cnt_t = jnp.sum((et == t).astype(F32))
        row_t = jnp.sum(etab * (rid == t).astype(F32), axis=0, keepdims=True)
        loop_row = loop_row + cnt_t * row_t
    loop_row = loop_row / 180000.0
    tab6 = etab * (rid < 4).astype(F32)
    tab6 = tab6 + (rid == 4).astype(F32) * ep_row
    tab6 = tab6 + (rid == 5).astype(F32) * loop_row
    tab6_ref[...] = tab6


BF16 = jnp.bfloat16


def _ln_kernel(x_ref, b_ref, o_ref, obf_ref):
    xn = _seg_ln(x_ref[...], b_ref[...])
    o_ref[...] = xn
    obf_ref[...] = xn.astype(BF16)


def _gen_kernel(xn_ref, tab6_ref, src_ref, dst_ref, code_ref, agg_ref):
    @pl.when(pl.program_id(0) == 0)
    def _():
        agg_ref[...] = jnp.zeros_like(agg_ref)
    ms = _onehot_cols(src_ref[0], NP, BF16)
    md = _onehot_cols(dst_ref[0], NP, BF16)
    xg = jnp.dot(ms, xn_ref[...], preferred_element_type=F32)
    xg = xg + jnp.dot(_onehot_cols(code_ref[0], 8), tab6_ref[...],
                      preferred_element_type=F32)
    msg = (jnp.maximum(xg, 0.0) + 1e-7).astype(BF16)
    cd = (((0,), (0,)), ((), ()))
    agg_ref[...] += jax.lax.dot_general(md, msg, cd, preferred_element_type=F32)


def _proj_kernel(xn_ref, agg_ref, tab6_ref,
                 wg_ref, bg_ref, wq_ref, bq_ref, wk_ref, bk_ref,
                 wv_ref, bv_ref, ws_ref, bs_ref, we_ref, be_ref,
                 qn_ref, kn_ref, vn_ref, skip_ref, ketab_ref):
    xn = xn_ref[...]
    gen = jnp.dot(xn + agg_ref[...], wg_ref[...], preferred_element_type=F32) + bg_ref[...]
    x2 = jnp.concatenate([xn, gen], axis=1)
    qn_ref[...] = (jnp.dot(x2, wq_ref[...], preferred_element_type=F32) + bq_ref[...]).astype(BF16)
    kn_ref[...] = (jnp.dot(x2, wk_ref[...], preferred_element_type=F32) + bk_ref[...]).astype(BF16)
    vn_ref[...] = (jnp.dot(x2, wv_ref[...], preferred_element_type=F32) + bv_ref[...]).astype(BF16)
    skip_ref[...] = jnp.dot(x2, ws_ref[...], preferred_element_type=F32) + bs_ref[...]
    ketab_ref[...] = jnp.dot(tab6_ref[...], we_ref[...], preferred_element_type=F32) + be_ref[...]


def _att_kernel(qn_ref, kn_ref, vn_ref, ketab_ref, src_ref, dst_ref, code_ref, att_ref):
    @pl.when(pl.program_id(0) == 0)
    def _():
        att_ref[...] = jnp.zeros_like(att_ref)
    ms = _onehot_cols(src_ref[0], NP, BF16)
    md = _onehot_cols(dst_ref[0], NP, BF16)
    ke = jnp.dot(_onehot_cols(code_ref[0], 8), ketab_ref[...],
                 preferred_element_type=F32)
    k_e = jnp.dot(ms, kn_ref[...], preferred_element_type=F32) + ke
    v_e = jnp.dot(ms, vn_ref[...], preferred_element_type=F32) + ke
    q_e = jnp.dot(md, qn_ref[...], preferred_element_type=F32)
    l1 = jnp.sum(q_e[:, :EMB] * k_e[:, :EMB], axis=1, keepdims=True) * 0.125
    l2 = jnp.sum(q_e[:, EMB:] * k_e[:, EMB:], axis=1, keepdims=True) * 0.125
    ex1 = jnp.exp(l1)
    ex2 = jnp.exp(l2)
    scat = jnp.concatenate([ex1 * v_e[:, :EMB], ex2 * v_e[:, EMB:], ex1, ex2],
                           axis=1).astype(BF16)
    cd = (((0,), (0,)), ((), ()))
    att_ref[...] += jax.lax.dot_general(md, scat, cd, preferred_element_type=F32)


def _post_kernel(att_ref, skip_ref, xres_ref, b_ref,
                 wl_ref, bl_ref, w1_ref, b1_ref, w2_ref, b2_ref, o_ref):
    att = att_ref[...]
    o1 = att[:, 0:EMB] / (att[:, 2 * EMB:2 * EMB + 1] + 1e-16)
    o2 = att[:, EMB:2 * EMB] / (att[:, 2 * EMB + 1:2 * EMB + 2] + 1e-16)
    out = jnp.concatenate([o1, o2], axis=1) + skip_ref[...]
    l_h = jnp.dot(out, wl_ref[...], preferred_element_type=F32) + bl_ref[...]
    h = _seg_ln(l_h, b_ref[...])
    hh = jnp.dot(h, w1_ref[...], preferred_element_type=F32) + b1_ref[...]
    hh = jnp.where(hh >= 0, hh, 0.01 * hh)
    ff = jnp.dot(hh, w2_ref[...], preferred_element_type=F32) + b2_ref[...]
    o_ref[...] = xres_ref[...] + ff


def _glob_heads_kernel(h_ref, breal_ref,
                       iw1, ib1, iw2, ib2, nw1, nb1, nw2, nb2, ew1, eb1, ew2, eb2,
                       o_init, o_nodelv, o_stop):
    S = _onehot_cols(breal_ref[...], NG)
    h = h_ref[...]
    ones = jnp.ones((NP, 1), F32)
    cd = (((0,), (0,)), ((), ()))
    cnt = jnp.maximum(jax.lax.dot_general(S, ones, cd, preferred_element_type=F32), 1.0)
    sums = jax.lax.dot_general(S, h, cd, preferred_element_type=F32)
    glob = sums / cnt + h_ref[N:NAUG, :]

    def m2(x, W1, b1, W2, b2):
        t = jnp.dot(x, W1[...], preferred_element_type=F32) + b1[...]
        t = jnp.where(t >= 0, t, 0.01 * t)
        return jnp.dot(t, W2[...], preferred_element_type=F32) + b2[...]
    o_init[...] = m2(glob, iw1, ib1, iw2, ib2)
    o_nodelv[...] = m2(glob, nw1, nb1, nw2, nb2)
    o_stop[...] = m2(glob, ew1, eb1, ew2, eb2)


def _ne_kernel(h_ref, a_ref, b_ref, tw1, tb1, tw2, tb2, o_ref):
    m = _onehot_cols(a_ref[0], NP, BF16) + _onehot_cols(b_ref[0], NP, BF16)
    ne = jnp.dot(m, h_ref[...], preferred_element_type=F32)
    t = jnp.dot(ne, tw1[...], preferred_element_type=F32) + tb1[...]
    t = jnp.where(t >= 0, t, 0.01 * t)
    o_ref[0] = jnp.dot(t, tw2[...], preferred_element_type=F32) + tb2[...]


def _pad_rows(x, rows):
    return jnp.pad(x, ((0, rows - x.shape[0]), (0, 0)))


def _col3(idx, chunk, pad_val):
    e = idx.shape[0]
    nch = -(-e // chunk)
    p = jnp.full((nch * chunk,), pad_val, jnp.int32).at[:e].set(idx.astype(jnp.int32))
    return p.reshape(nch, chunk, 1), nch


def _edge_call(kfn, n_out_lanes, nch, full_ins, idx_ins):
    grid = (nch,)
    in_specs = ([pl.BlockSpec(a.shape, lambda i: (0,) * a.ndim) for a in full_ins]
                + [pl.BlockSpec((1, a.shape[1], 1), lambda i: (i, 0, 0)) for a in idx_ins])
    return pl.pallas_call(
        kfn,
        grid=grid,
        in_specs=in_specs,
        out_specs=pl.BlockSpec((NP, n_out_lanes), lambda i: (0, 0)),
        out_shape=jax.ShapeDtypeStruct((NP, n_out_lanes), F32),
        compiler_params=pltpu.CompilerParams(vmem_limit_bytes=120 * 1024 * 1024),
    )(*full_ins, *idx_ins)


def kernel(params, node_type, node_state_type, frontier_order, edge_type, edge_index, batch, non_edge_index):
    b2 = lambda b: b.reshape(1, -1)
    i32 = lambda a: a.astype(jnp.int32)

    ntab = _pad_rows(params['node_type_emb'], 24)
    stab = _pad_rows(params['node_state_emb'], 8)
    etab = _pad_rows(params['edge_type_emb'], 8)
    x0, tab6 = pl.pallas_call(
        _setup_kernel,
        out_shape=[jax.ShapeDtypeStruct((N, EMB), F32),
                   jax.ShapeDtypeStruct((8, EMB), F32)],
    )(i32(node_type).reshape(N, 1), i32(node_state_type).reshape(N, 1),
      i32(frontier_order).reshape(N, 1),
      jnp.pad(i32(edge_type), (0, 160768 - edge_type.shape[0]),
              constant_values=100).reshape(1256, 128),
      ntab, stab, etab)

    cond = jnp.broadcast_to(params['virtual_emb'][0], (NG, EMB))
    x_aug = _pad_rows(jnp.concatenate([x0, cond], axis=0), NP)
    u = jnp.arange(N, dtype=jnp.int32)
    v = i32(batch) + N
    loop = jnp.arange(NAUG, dtype=jnp.int32)
    src = jnp.concatenate([i32(edge_index[0]), u, v, loop])
    dst = jnp.concatenate([i32(edge_index[1]), v, u, loop])
    codes = jnp.concatenate([i32(edge_type), jnp.full((2 * N,), 4, jnp.int32),
                             jnp.full((NAUG,), 5, jnp.int32)])
    src3, nch = _col3(src, CH, NP)
    dst3, _ = _col3(dst, CH, NP)
    cod3, _ = _col3(codes, CH, 0)
    batch_col = jnp.concatenate([i32(batch), jnp.arange(NG, dtype=jnp.int32),
                                 jnp.full((NP - NAUG,), NG, jnp.int32)]).reshape(NP, 1)

    ln_call = pl.pallas_call(
        _ln_kernel, out_shape=[jax.ShapeDtypeStruct((NP, EMB), F32),
                               jax.ShapeDtypeStruct((NP, EMB), BF16)])
    proj_call = pl.pallas_call(
        _proj_kernel,
        out_shape=[jax.ShapeDtypeStruct((NP, 2 * EMB), BF16)] * 3
        + [jax.ShapeDtypeStruct((NP, 2 * EMB), F32),
           jax.ShapeDtypeStruct((8, 2 * EMB), F32)])
    post_call = pl.pallas_call(
        _post_kernel, out_shape=jax.ShapeDtypeStruct((NP, EMB), F32))

    h = x_aug
    for p in params['layers']:
        xn, xn_bf = ln_call(h, batch_col)
        agg = _edge_call(_gen_kernel, EMB, nch, [xn_bf, tab6], [src3, dst3, cod3])
        qn, kn, vn, skip, ketab = proj_call(
            xn, agg, tab6,
            p['W_gen'], b2(p['b_gen']), p['Wq'], b2(p['bq']), p['Wk'], b2(p['bk']),
            p['Wv'], b2(p['bv']), p['Wskip'], b2(p['bskip']), p['We'], b2(p['be']))
        att = _edge_call(_att_kernel, 2 * EMB + 2, nch,
                         [qn, kn, vn, ketab], [src3, dst3, cod3])
        h = post_call(att, skip, h, batch_col,
                      p['Wl'], b2(p['bl']), p['W1'], b2(p['b1']), p['W2'], b2(p['b2']))

    breal_col = jnp.concatenate([i32(batch), jnp.full((NP - N,), NG, jnp.int32)]).reshape(NP, 1)
    init_f, nodelv_f, stop_f = pl.pallas_call(
        _glob_heads_kernel,
        out_shape=[jax.ShapeDtypeStruct((NG, 16), F32),
                   jax.ShapeDtypeStruct((NG, 65), F32),
                   jax.ShapeDtypeStruct((NG, 1), F32)],
    )(h, breal_col,
      params['init_W1'], b2(params['init_b1']), params['init_W2'], b2(params['init_b2']),
      params['nodelv_W1'], b2(params['nodelv_b1']), params['nodelv_W2'], b2(params['nodelv_b2']),
      params['e1_W1'], b2(params['e1_b1']), params['e1_W2'], b2(params['e1_b2']))

    a3, nne = _col3(i32(non_edge_index[0]), CH2, NP)
    bb3, _ = _col3(i32(non_edge_index[1]), CH2, NP)
    tgt_f = pl.pallas_call(
        _ne_kernel,
        grid=(nne,),
        in_specs=[pl.BlockSpec((NP, EMB), lambda i: (0, 0)),
                  pl.BlockSpec((1, CH2, 1), lambda i: (i, 0, 0)),
                  pl.BlockSpec((1, CH2, 1), lambda i: (i, 0, 0)),
                  pl.BlockSpec((EMB, EMB), lambda i: (0, 0)),
                  pl.BlockSpec((1, EMB), lambda i: (0, 0)),
                  pl.BlockSpec((EMB, 4), lambda i: (0, 0)),
                  pl.BlockSpec((1, 4), lambda i: (0, 0))],
        out_specs=pl.BlockSpec((1, CH2, 4), lambda i: (i, 0, 0)),
        out_shape=jax.ShapeDtypeStruct((nne, CH2, 4), F32),
    )(h.astype(BF16), a3, bb3,
      params['e2_W1'], b2(params['e2_b1']), params['e2_W2'], b2(params['e2_b2']))

    i = NG // 3
    j = 2 * NG // 3
    nne_total = non_edge_index.shape[1]
    return jnp.concatenate([
        init_f[:i].reshape(-1), nodelv_f[i:j].reshape(-1), stop_f[j:].reshape(-1),
        tgt_f.reshape(-1, 4)[:nne_total].reshape(-1)], axis=0)
